# baseline (speedup 1.0000x reference)
import functools

import jax
import jax.numpy as jnp
from jax import lax
from jax.experimental import pallas as pl
from jax.experimental.pallas import tpu as pltpu
from jax.experimental.pallas import tpu_sc as plsc

VOCAB = 100000
DIM = 300
BATCH = 4096
BAG = 50

NUM_CORES = 2
NUM_SUBCORES = 16
NW = NUM_CORES * NUM_SUBCORES
BPW = BATCH // NW
LANES = 16
PACKED = 256
NQ = 128 // LANES
TAIL = DIM - PACKED
NT = 3
DIM_PAD = PACKED + NT * LANES
SCALE = 1.0 / BAG

_mesh = plsc.VectorSubcoreMesh(core_axis_name="c", subcore_axis_name="s")


@functools.partial(
    pl.kernel,
    mesh=_mesh,
    out_type=jax.ShapeDtypeStruct((BATCH, DIM_PAD), jnp.float32),
    scratch_types=[
        pltpu.VMEM((BPW, BAG), jnp.int32),
        pltpu.VMEM((2, BAG, 128), jnp.int32),
        pltpu.VMEM((2, BAG, 128), jnp.float32),
        pltpu.VMEM((BPW, DIM_PAD), jnp.float32),
        pltpu.SemaphoreType.DMA,
        pltpu.SemaphoreType.DMA,
    ],
)
def _bow_sc(idx_hbm, tw_hbm, tail_hbm, out_hbm, idx_v, roww_v, rowt_v, out_v,
            sem0, sem1):
    wid = lax.axis_index("s") * NUM_CORES + lax.axis_index("c")
    base = wid * BPW
    sems = (sem0, sem1)

    pltpu.sync_copy(idx_hbm.at[pl.ds(base, BPW)], idx_v)

    def issue(g, buf):
        idx = idx_v.at[g]
        pltpu.async_copy(tw_hbm.at[idx], roww_v.at[buf], sems[buf])
        pltpu.async_copy(tail_hbm.at[idx], rowt_v.at[buf], sems[buf])

    def wait_buf(g, buf):
        idx = idx_v.at[g]
        pltpu.make_async_copy(tw_hbm.at[idx], roww_v.at[buf], sems[buf]).wait()
        pltpu.make_async_copy(tail_hbm.at[idx], rowt_v.at[buf], sems[buf]).wait()

    hi_mask = jnp.full((LANES,), -65536, jnp.int32)

    def reduce_bag(g, buf):
        def body(r, accs):
            new = []
            for q in range(NQ):
                w = roww_v[buf, r, pl.ds(LANES * q, LANES)]
                lo = lax.bitcast_convert_type(lax.shift_left(w, 16), jnp.float32)
                hi = lax.bitcast_convert_type(
                    lax.bitwise_and(w, hi_mask), jnp.float32
                )
                new.append(accs[2 * q] + lo)
                new.append(accs[2 * q + 1] + hi)
            for t in range(NT):
                v = rowt_v[buf, r, pl.ds(LANES * t, LANES)]
                new.append(accs[2 * NQ + t] + v)
            return tuple(new)

        zero = jnp.zeros((LANES,), jnp.float32)
        accs = lax.fori_loop(0, BAG, body, (zero,) * (2 * NQ + NT))
        for q in range(NQ):
            out_v[g, pl.ds(LANES * q, LANES)] = accs[2 * q] * SCALE
            out_v[g, pl.ds(128 + LANES * q, LANES)] = accs[2 * q + 1] * SCALE
        for t in range(NT):
            out_v[g, pl.ds(PACKED + LANES * t, LANES)] = accs[2 * NQ + t] * SCALE

    issue(0, 0)

    def pair_body(p, carry):
        for h in range(2):
            g = p * 2 + h

            @pl.when(g + 1 < BPW)
            def _():
                issue(g + 1, 1 - h)

            wait_buf(g, h)
            reduce_bag(g, h)
        return carry

    lax.fori_loop(0, BPW // 2, pair_body, 0)
    pltpu.sync_copy(out_v, out_hbm.at[pl.ds(base, BPW)])


def kernel(indices, table):
    idx = jnp.asarray(indices, jnp.int32)
    lo = lax.bitcast_convert_type(
        table[:, :128].astype(jnp.bfloat16), jnp.uint16
    ).astype(jnp.uint32)
    hi = lax.bitcast_convert_type(
        table[:, 128:PACKED].astype(jnp.bfloat16), jnp.uint16
    ).astype(jnp.uint32)
    tw = lax.bitcast_convert_type(
        lax.bitwise_or(lo, lax.shift_left(hi, jnp.uint32(16))), jnp.int32
    )
    tail = jnp.pad(table[:, PACKED:], ((0, 0), (0, 128 - TAIL)))
    return _bow_sc(idx, tw, tail)[:, :DIM]


# speedup vs baseline: 2.2448x; 2.2448x over previous
import functools

import jax
import jax.numpy as jnp
from jax import lax
from jax.experimental import pallas as pl
from jax.experimental.pallas import tpu as pltpu
from jax.experimental.pallas import tpu_sc as plsc

VOCAB = 100000
DIM = 300
BATCH = 4096
BAG = 50

NUM_CORES = 2
NUM_SUBCORES = 16
NW = NUM_CORES * NUM_SUBCORES
BPW = BATCH // NW
LANES = 16
TILE = 128
TAIL = DIM - 2 * TILE

_FULL_STARTS = [16 * i for i in range(TILE // 16)]
_TAIL_STARTS = [0, 16, 32]
_CHUNKS = (
    [(0, s) for s in _FULL_STARTS]
    + [(1, s) for s in _FULL_STARTS]
    + [(2, s) for s in _TAIL_STARTS]
)
NCHUNK = len(_CHUNKS)
DIM_PAD = 3 * TILE
SCALE = 1.0 / BAG

_mesh = plsc.VectorSubcoreMesh(core_axis_name="c", subcore_axis_name="s")


@functools.partial(
    pl.kernel,
    mesh=_mesh,
    out_type=jax.ShapeDtypeStruct((BATCH, DIM_PAD), jnp.float32),
    scratch_types=[
        pltpu.VMEM((BPW, BAG), jnp.int32),
        pltpu.VMEM((2, 3, BAG, TILE), jnp.float32),
        pltpu.VMEM((BPW, DIM_PAD), jnp.float32),
        pltpu.SemaphoreType.DMA,
        pltpu.SemaphoreType.DMA,
    ],
)
def _bow_sc(idx_hbm, table_hbm, tail_hbm, out_hbm, idx_v, rows_v, out_v,
            sem0, sem1):
    wid = lax.axis_index("s") * NUM_CORES + lax.axis_index("c")
    base = wid * BPW
    sems = (sem0, sem1)

    pltpu.sync_copy(idx_hbm.at[pl.ds(base, BPW)], idx_v)

    def srcs(g):
        idx = idx_v.at[g]
        return (
            table_hbm.at[idx, pl.ds(0, TILE)],
            table_hbm.at[idx, pl.ds(TILE, TILE)],
            tail_hbm.at[idx],
        )

    def issue(g, buf):
        for j, src in enumerate(srcs(g)):
            pltpu.async_copy(src, rows_v.at[buf, j], sems[buf])

    def wait_buf(g, buf):
        for j, src in enumerate(srcs(g)):
            pltpu.make_async_copy(src, rows_v.at[buf, j], sems[buf]).wait()

    def reduce_bag(g, buf):
        def body(r, accs):
            return tuple(
                accs[i] + rows_v[buf, j, r, pl.ds(s, LANES)]
                for i, (j, s) in enumerate(_CHUNKS)
            )

        zero = jnp.zeros((LANES,), jnp.float32)
        accs = lax.fori_loop(0, BAG, body, (zero,) * NCHUNK)
        for i, (j, s) in enumerate(_CHUNKS):
            out_v[g, pl.ds(j * TILE + s, LANES)] = accs[i] * SCALE

    issue(0, 0)

    def pair_body(p, carry):
        for h in range(2):
            g = p * 2 + h

            @pl.when(g + 1 < BPW)
            def _():
                issue(g + 1, 1 - h)

            wait_buf(g, h)
            reduce_bag(g, h)
        return carry

    lax.fori_loop(0, BPW // 2, pair_body, 0)
    pltpu.sync_copy(out_v, out_hbm.at[pl.ds(base, BPW)])


def kernel(indices, table):
    idx = jnp.asarray(indices, jnp.int32)
    tail = jnp.pad(table[:, 2 * TILE:], ((0, 0), (0, TILE - TAIL)))
    return _bow_sc(idx, table, tail)[:, :DIM]
